# Initial kernel scaffold; baseline (speedup 1.0000x reference)
import jax
import jax.numpy as jnp
from jax.experimental import pallas as pl


def kernel(x, pos, batch, x_skip, pos_skip, batch_skip, W1, b1, W2, b2):
    raise NotImplementedError("write your pallas kernel here")


# speedup vs baseline: 10.4819x; 10.4819x over previous
import functools

import jax
import jax.numpy as jnp
from jax import lax
from jax.experimental import pallas as pl
from jax.experimental.pallas import tpu as pltpu
from jax.experimental.pallas import tpu_sc as plsc


def _knn_body(ps_ref, posT_ref, idx_ref, wn_ref, *, n1):
    q = ps_ref[:]
    kT = posT_ref[:]
    qn = jnp.sum(q * q, axis=1, keepdims=True)
    kn = jnp.sum(kT * kT, axis=0, keepdims=True)
    dt = lax.dot_general(q, kT, (((1,), (0,)), ((), ())),
                         preferred_element_type=jnp.float32)
    d2 = qn + kn - 2.0 * dt
    iota = lax.broadcasted_iota(jnp.int32, d2.shape, 1)
    inf = jnp.float32(jnp.inf)

    def extract(d):
        m = jnp.min(d, axis=1, keepdims=True)
        im = jnp.min(jnp.where(d <= m, iota, n1), axis=1, keepdims=True)
        return m, im

    m1, i1 = extract(d2)
    d2 = jnp.where(iota == i1, inf, d2)
    m2, i2 = extract(d2)
    d2 = jnp.where(iota == i2, inf, d2)
    m3, i3 = extract(d2)

    w1 = 1.0 / jnp.maximum(jnp.maximum(m1, 0.0), 1e-16)
    w2 = 1.0 / jnp.maximum(jnp.maximum(m2, 0.0), 1e-16)
    w3 = 1.0 / jnp.maximum(jnp.maximum(m3, 0.0), 1e-16)
    ws = w1 + w2 + w3
    idx_ref[:] = jnp.concatenate([i1, i2, i3], axis=1)
    wn_ref[:] = jnp.concatenate([w1 / ws, w2 / ws, w3 / ws], axis=1)


def _knn_call(ps_pad, posT):
    n2 = ps_pad.shape[0]
    n1 = posT.shape[1]
    r = 256
    return pl.pallas_call(
        functools.partial(_knn_body, n1=n1),
        grid=(n2 // r,),
        in_specs=[
            pl.BlockSpec((r, 8), lambda i: (i, 0)),
            pl.BlockSpec((8, n1), lambda i: (0, 0)),
        ],
        out_specs=[
            pl.BlockSpec((r, 3), lambda i: (i, 0)),
            pl.BlockSpec((r, 3), lambda i: (i, 0)),
        ],
        out_shape=[
            jax.ShapeDtypeStruct((n2, 3), jnp.int32),
            jax.ShapeDtypeStruct((n2, 3), jnp.float32),
        ],
    )(ps_pad, posT)


def _gather_call(x, idx_flat):
    n_rows = idx_flat.shape[0]
    d = x.shape[1]
    info = plsc.get_sparse_core_info()
    nw = info.num_cores * info.num_subcores
    per_w = n_rows // nw
    qc3 = 96
    n_chunks = per_w // qc3

    mesh = plsc.VectorSubcoreMesh(core_axis_name="c", subcore_axis_name="s")

    @functools.partial(
        pl.kernel,
        mesh=mesh,
        out_type=jax.ShapeDtypeStruct((n_rows, d), jnp.float32),
        scratch_types=[
            pltpu.VMEM((qc3,), jnp.int32),
            pltpu.VMEM((qc3, d), jnp.float32),
            pltpu.SemaphoreType.DMA,
        ],
    )
    def gather_kernel(x_hbm, idx_hbm, out_hbm, idx_v, rows_v, sem):
        wid = lax.axis_index("s") * info.num_cores + lax.axis_index("c")
        base = wid * per_w

        def chunk(i, carry):
            b = base + i * qc3
            pltpu.sync_copy(idx_hbm.at[pl.ds(b, qc3)], idx_v)
            pltpu.async_copy(x_hbm.at[idx_v], rows_v, sem).wait()
            pltpu.sync_copy(rows_v, out_hbm.at[pl.ds(b, qc3)])
            return carry

        lax.fori_loop(0, n_chunks, chunk, 0)

    return gather_kernel(x, idx_flat)


def _mlp_body(f_ref, wn_ref, xs_ref, w1a_ref, w1b_ref, b1_ref, w2_ref, b2_ref,
              out_ref):
    wn = wn_ref[:]
    y = (f_ref[:, 0, :] * wn[:, 0:1]
         + f_ref[:, 1, :] * wn[:, 1:2]
         + f_ref[:, 2, :] * wn[:, 2:3])
    dn = (((1,), (0,)), ((), ()))
    h = jnp.maximum(
        lax.dot_general(y, w1a_ref[:], dn, preferred_element_type=jnp.float32)
        + lax.dot_general(xs_ref[:], w1b_ref[:], dn,
                          preferred_element_type=jnp.float32)
        + b1_ref[:], 0.0)
    out_ref[:] = jnp.maximum(
        lax.dot_general(h, w2_ref[:], dn, preferred_element_type=jnp.float32)
        + b2_ref[:], 0.0)


def _mlp_call(feats, wn, x_skip, w1a, w1b, b1r, w2, b2r):
    n2 = feats.shape[0]
    d_in = feats.shape[2]
    d_skip = x_skip.shape[1]
    d_hid = w2.shape[0]
    d_out = w2.shape[1]
    r = 512
    return pl.pallas_call(
        _mlp_body,
        grid=(n2 // r,),
        in_specs=[
            pl.BlockSpec((r, 3, d_in), lambda i: (i, 0, 0)),
            pl.BlockSpec((r, 3), lambda i: (i, 0)),
            pl.BlockSpec((r, d_skip), lambda i: (i, 0)),
            pl.BlockSpec((d_in, d_hid), lambda i: (0, 0)),
            pl.BlockSpec((d_skip, d_hid), lambda i: (0, 0)),
            pl.BlockSpec((1, d_hid), lambda i: (0, 0)),
            pl.BlockSpec((d_hid, d_out), lambda i: (0, 0)),
            pl.BlockSpec((1, d_out), lambda i: (0, 0)),
        ],
        out_specs=pl.BlockSpec((r, d_out), lambda i: (i, 0)),
        out_shape=jax.ShapeDtypeStruct((n2, d_out), jnp.float32),
    )(feats, wn, x_skip, w1a, w1b, b1r, w2, b2r)


def kernel(x, pos, batch, x_skip, pos_skip, batch_skip, W1, b1, W2, b2):
    n1, d_in = x.shape
    n2 = pos_skip.shape[0]
    ps_pad = jnp.pad(pos_skip, ((0, 0), (0, 8 - pos_skip.shape[1])))
    posT = jnp.pad(pos, ((0, 0), (0, 8 - pos.shape[1]))).T
    idx, wn = _knn_call(ps_pad, posT)
    feats = _gather_call(x, idx.reshape(-1))
    out = _mlp_call(feats.reshape(n2, 3, d_in), wn, x_skip,
                    W1[:d_in], W1[d_in:], b1.reshape(1, -1),
                    W2, b2.reshape(1, -1))
    return (out, pos_skip, batch_skip)


# speedup vs baseline: 11.0507x; 1.0543x over previous
import functools

import jax
import jax.numpy as jnp
from jax import lax
from jax.experimental import pallas as pl
from jax.experimental.pallas import tpu as pltpu
from jax.experimental.pallas import tpu_sc as plsc


def _knn_body(ps_ref, pos_ref, idx_ref, wn_ref, *, n1):
    q = ps_ref[:]
    k = pos_ref[:]
    dn = (((1,), (1,)), ((), ()))
    kn = lax.dot_general(jnp.ones((1, 3), jnp.float32), k * k, dn,
                         preferred_element_type=jnp.float32)
    dt = lax.dot_general(q, k, dn, preferred_element_type=jnp.float32)
    s = kn - 2.0 * dt
    iota = lax.broadcasted_iota(jnp.int32, s.shape, 1)
    inf = jnp.float32(jnp.inf)

    def extract(d):
        m = jnp.min(d, axis=1, keepdims=True)
        im = jnp.min(jnp.where(d <= m, iota, n1), axis=1, keepdims=True)
        return m, im

    m1, i1 = extract(s)
    s = jnp.where(iota == i1, inf, s)
    m2, i2 = extract(s)
    s = jnp.where(iota == i2, inf, s)
    m3, i3 = extract(s)

    qn = jnp.sum(q * q, axis=1, keepdims=True)
    w1 = 1.0 / jnp.maximum(jnp.maximum(m1 + qn, 0.0), 1e-16)
    w2 = 1.0 / jnp.maximum(jnp.maximum(m2 + qn, 0.0), 1e-16)
    w3 = 1.0 / jnp.maximum(jnp.maximum(m3 + qn, 0.0), 1e-16)
    ws = w1 + w2 + w3
    idx_ref[:] = jnp.concatenate([i1, i2, i3], axis=1)
    wn_ref[:] = jnp.concatenate([w1 / ws, w2 / ws, w3 / ws], axis=1)


def _knn_call(pos_skip, pos):
    n2 = pos_skip.shape[0]
    n1 = pos.shape[0]
    r = 512
    return pl.pallas_call(
        functools.partial(_knn_body, n1=n1),
        grid=(n2 // r,),
        in_specs=[
            pl.BlockSpec((r, 3), lambda i: (i, 0)),
            pl.BlockSpec((n1, 3), lambda i: (0, 0)),
        ],
        out_specs=[
            pl.BlockSpec((r, 3), lambda i: (i, 0)),
            pl.BlockSpec((r, 3), lambda i: (i, 0)),
        ],
        out_shape=[
            jax.ShapeDtypeStruct((n2, 3), jnp.int32),
            jax.ShapeDtypeStruct((n2, 3), jnp.float32),
        ],
    )(pos_skip, pos)


_QC3 = 96
_NBUF = 4


def _gather_call(x, idx2d):
    n_chunks_total, qc3 = idx2d.shape
    n_rows = n_chunks_total * qc3
    d = x.shape[1]
    info = plsc.get_sparse_core_info()
    nw = info.num_cores * info.num_subcores
    cpw = n_chunks_total // nw

    mesh = plsc.VectorSubcoreMesh(core_axis_name="c", subcore_axis_name="s")

    @functools.partial(
        pl.kernel,
        mesh=mesh,
        out_type=jax.ShapeDtypeStruct((n_rows, d), jnp.float32),
        scratch_types=(
            [pltpu.VMEM((cpw, qc3), jnp.int32)]
            + [pltpu.VMEM((qc3, d), jnp.float32) for _ in range(_NBUF)]
            + [pltpu.SemaphoreType.DMA for _ in range(_NBUF)]
        ),
    )
    def gather_kernel(x_hbm, idx_hbm, out_hbm, idx_v, *bufs_and_sems):
        rows = bufs_and_sems[:_NBUF]
        sems = bufs_and_sems[_NBUF:]
        wid = lax.axis_index("s") * info.num_cores + lax.axis_index("c")
        base_c = wid * cpw
        base_r = base_c * qc3
        pltpu.sync_copy(idx_hbm.at[pl.ds(base_c, cpw)], idx_v)
        for j in range(min(_NBUF, cpw)):
            pltpu.async_copy(x_hbm.at[idx_v.at[j]], rows[j], sems[j])
        for i in range(cpw):
            b = i % _NBUF
            pltpu.make_async_copy(x_hbm.at[idx_v.at[i]], rows[b],
                                  sems[b]).wait()
            pltpu.sync_copy(rows[b], out_hbm.at[pl.ds(base_r + i * qc3, qc3)])
            nxt = i + _NBUF
            if nxt < cpw:
                pltpu.async_copy(x_hbm.at[idx_v.at[nxt]], rows[b], sems[b])

    return gather_kernel(x, idx2d)


def _mlp_body(f_ref, wn_ref, xs_ref, w1_ref, b1_ref, w2_ref, b2_ref, out_ref):
    d_in = f_ref.shape[2]
    wn = wn_ref[:]
    y = (f_ref[:, 0, :] * wn[:, 0:1]
         + f_ref[:, 1, :] * wn[:, 1:2]
         + f_ref[:, 2, :] * wn[:, 2:3])
    dn = (((1,), (0,)), ((), ()))
    h = jnp.maximum(
        lax.dot_general(y, w1_ref[0:d_in, :], dn,
                        preferred_element_type=jnp.float32)
        + lax.dot_general(xs_ref[:], w1_ref[d_in:, :], dn,
                          preferred_element_type=jnp.float32)
        + b1_ref[:], 0.0)
    out_ref[:] = jnp.maximum(
        lax.dot_general(h, w2_ref[:], dn, preferred_element_type=jnp.float32)
        + b2_ref[:], 0.0)


def _mlp_call(feats, wn, x_skip, w1, b1r, w2, b2r):
    n2 = feats.shape[0]
    d_in = feats.shape[2]
    d_skip = x_skip.shape[1]
    d_tot = w1.shape[0]
    d_hid = w1.shape[1]
    d_out = w2.shape[1]
    r = 512
    return pl.pallas_call(
        _mlp_body,
        grid=(n2 // r,),
        in_specs=[
            pl.BlockSpec((r, 3, d_in), lambda i: (i, 0, 0)),
            pl.BlockSpec((r, 3), lambda i: (i, 0)),
            pl.BlockSpec((r, d_skip), lambda i: (i, 0)),
            pl.BlockSpec((d_tot, d_hid), lambda i: (0, 0)),
            pl.BlockSpec((1, d_hid), lambda i: (0, 0)),
            pl.BlockSpec((d_hid, d_out), lambda i: (0, 0)),
            pl.BlockSpec((1, d_out), lambda i: (0, 0)),
        ],
        out_specs=pl.BlockSpec((r, d_out), lambda i: (i, 0)),
        out_shape=jax.ShapeDtypeStruct((n2, d_out), jnp.float32),
    )(feats, wn, x_skip, w1, b1r, w2, b2r)


def kernel(x, pos, batch, x_skip, pos_skip, batch_skip, W1, b1, W2, b2):
    n1, d_in = x.shape
    n2 = pos_skip.shape[0]
    idx, wn = _knn_call(pos_skip, pos)
    feats = _gather_call(x, idx.reshape(-1).reshape(-1, _QC3))
    out = _mlp_call(feats.reshape(n2, 3, d_in), wn, x_skip,
                    W1, b1.reshape(1, -1), W2, b2.reshape(1, -1))
    return (out, pos_skip, batch_skip)


# speedup vs baseline: 11.2057x; 1.0140x over previous
import functools

import jax
import jax.numpy as jnp
from jax import lax
from jax.experimental import pallas as pl
from jax.experimental.pallas import tpu as pltpu
from jax.experimental.pallas import tpu_sc as plsc


def _knn_body(ps_ref, pos_ref, idx_ref, wn_ref, *, n1):
    q = ps_ref[:]
    kT = pos_ref[:]
    kn = jnp.sum(kT * kT, axis=0, keepdims=True)
    dt = lax.dot_general(q, kT, (((1,), (0,)), ((), ())),
                         preferred_element_type=jnp.float32)
    s = kn - 2.0 * dt
    iota = lax.broadcasted_iota(jnp.int32, s.shape, 1)
    inf = jnp.float32(jnp.inf)

    def extract(d):
        m = jnp.min(d, axis=1, keepdims=True)
        im = jnp.min(jnp.where(d <= m, iota, n1), axis=1, keepdims=True)
        return m, im

    m1, i1 = extract(s)
    s = jnp.where(iota == i1, inf, s)
    m2, i2 = extract(s)
    s = jnp.where(iota == i2, inf, s)
    m3, i3 = extract(s)

    qn = jnp.sum(q * q, axis=1, keepdims=True)
    w1 = 1.0 / jnp.maximum(jnp.maximum(m1 + qn, 0.0), 1e-16)
    w2 = 1.0 / jnp.maximum(jnp.maximum(m2 + qn, 0.0), 1e-16)
    w3 = 1.0 / jnp.maximum(jnp.maximum(m3 + qn, 0.0), 1e-16)
    ws = w1 + w2 + w3
    idx_ref[:] = jnp.concatenate([i1, i2, i3], axis=1)
    wn_ref[:] = jnp.concatenate([w1 / ws, w2 / ws, w3 / ws], axis=1)


def _knn_call(ps_pad, posT):
    n2 = ps_pad.shape[0]
    n1 = posT.shape[1]
    r = 512
    return pl.pallas_call(
        functools.partial(_knn_body, n1=n1),
        grid=(n2 // r,),
        in_specs=[
            pl.BlockSpec((r, 8), lambda i: (i, 0)),
            pl.BlockSpec((8, n1), lambda i: (0, 0)),
        ],
        out_specs=[
            pl.BlockSpec((r, 3), lambda i: (i, 0)),
            pl.BlockSpec((r, 3), lambda i: (i, 0)),
        ],
        out_shape=[
            jax.ShapeDtypeStruct((n2, 3), jnp.int32),
            jax.ShapeDtypeStruct((n2, 3), jnp.float32),
        ],
    )(ps_pad, posT)


_QC3 = 96
_NBUF = 4


def _gather_call(x, idx2d):
    n_chunks_total, qc3 = idx2d.shape
    n_rows = n_chunks_total * qc3
    d = x.shape[1]
    info = plsc.get_sparse_core_info()
    nw = info.num_cores * info.num_subcores
    cpw = n_chunks_total // nw

    mesh = plsc.VectorSubcoreMesh(core_axis_name="c", subcore_axis_name="s")

    @functools.partial(
        pl.kernel,
        mesh=mesh,
        out_type=jax.ShapeDtypeStruct((n_rows, d), jnp.float32),
        scratch_types=(
            [pltpu.VMEM((cpw, qc3), jnp.int32)]
            + [pltpu.VMEM((qc3, d), jnp.float32) for _ in range(_NBUF)]
            + [pltpu.SemaphoreType.DMA for _ in range(_NBUF)]
        ),
    )
    def gather_kernel(x_hbm, idx_hbm, out_hbm, idx_v, *bufs_and_sems):
        rows = bufs_and_sems[:_NBUF]
        sems = bufs_and_sems[_NBUF:]
        wid = lax.axis_index("s") * info.num_cores + lax.axis_index("c")
        base_c = wid * cpw
        base_r = base_c * qc3
        pltpu.sync_copy(idx_hbm.at[pl.ds(base_c, cpw)], idx_v)
        for j in range(min(_NBUF, cpw)):
            pltpu.async_copy(x_hbm.at[idx_v.at[j]], rows[j], sems[j])
        for i in range(cpw):
            b = i % _NBUF
            pltpu.make_async_copy(x_hbm.at[idx_v.at[i]], rows[b],
                                  sems[b]).wait()
            pltpu.sync_copy(rows[b], out_hbm.at[pl.ds(base_r + i * qc3, qc3)])
            nxt = i + _NBUF
            if nxt < cpw:
                pltpu.async_copy(x_hbm.at[idx_v.at[nxt]], rows[b], sems[b])

    return gather_kernel(x, idx2d)


def _mlp_body(f_ref, wn_ref, xs_ref, w1_ref, b1_ref, w2_ref, b2_ref, out_ref):
    d_in = f_ref.shape[2]
    wn = wn_ref[:]
    y = (f_ref[:, 0, :] * wn[:, 0:1]
         + f_ref[:, 1, :] * wn[:, 1:2]
         + f_ref[:, 2, :] * wn[:, 2:3])
    dn = (((1,), (0,)), ((), ()))
    h = jnp.maximum(
        lax.dot_general(y, w1_ref[0:d_in, :], dn,
                        preferred_element_type=jnp.float32)
        + lax.dot_general(xs_ref[:], w1_ref[d_in:, :], dn,
                          preferred_element_type=jnp.float32)
        + b1_ref[:], 0.0)
    out_ref[:] = jnp.maximum(
        lax.dot_general(h, w2_ref[:], dn, preferred_element_type=jnp.float32)
        + b2_ref[:], 0.0)


def _mlp_call(feats, wn, x_skip, w1, b1r, w2, b2r):
    n2 = feats.shape[0]
    d_in = feats.shape[2]
    d_skip = x_skip.shape[1]
    d_tot = w1.shape[0]
    d_hid = w1.shape[1]
    d_out = w2.shape[1]
    r = 512
    return pl.pallas_call(
        _mlp_body,
        grid=(n2 // r,),
        in_specs=[
            pl.BlockSpec((r, 3, d_in), lambda i: (i, 0, 0)),
            pl.BlockSpec((r, 3), lambda i: (i, 0)),
            pl.BlockSpec((r, d_skip), lambda i: (i, 0)),
            pl.BlockSpec((d_tot, d_hid), lambda i: (0, 0)),
            pl.BlockSpec((1, d_hid), lambda i: (0, 0)),
            pl.BlockSpec((d_hid, d_out), lambda i: (0, 0)),
            pl.BlockSpec((1, d_out), lambda i: (0, 0)),
        ],
        out_specs=pl.BlockSpec((r, d_out), lambda i: (i, 0)),
        out_shape=jax.ShapeDtypeStruct((n2, d_out), jnp.float32),
    )(feats, wn, x_skip, w1, b1r, w2, b2r)


def kernel(x, pos, batch, x_skip, pos_skip, batch_skip, W1, b1, W2, b2):
    n1, d_in = x.shape
    n2 = pos_skip.shape[0]
    ps_pad = jnp.pad(pos_skip, ((0, 0), (0, 8 - pos_skip.shape[1])))
    posT = jnp.pad(pos, ((0, 0), (0, 8 - pos.shape[1]))).T
    idx, wn = _knn_call(ps_pad, posT)
    feats = _gather_call(x, idx.reshape(-1).reshape(-1, _QC3))
    out = _mlp_call(feats.reshape(n2, 3, d_in), wn, x_skip,
                    W1, b1.reshape(1, -1), W2, b2.reshape(1, -1))
    return (out, pos_skip, batch_skip)


# speedup vs baseline: 14.1147x; 1.2596x over previous
import functools

import jax
import jax.numpy as jnp
from jax import lax
from jax.experimental import pallas as pl
from jax.experimental.pallas import tpu as pltpu
from jax.experimental.pallas import tpu_sc as plsc


def _knn_body(ps_ref, pos_ref, idx_ref, wn_ref, *, n1):
    q = ps_ref[:]
    kT = pos_ref[:]
    kn = jnp.sum(kT * kT, axis=0, keepdims=True)
    dt = lax.dot_general(q, kT, (((1,), (0,)), ((), ())),
                         preferred_element_type=jnp.float32)
    s = kn - 2.0 * dt
    iota = lax.broadcasted_iota(jnp.int32, s.shape, 1)
    inf = jnp.float32(jnp.inf)

    def extract(d):
        m = jnp.min(d, axis=1, keepdims=True)
        im = jnp.min(jnp.where(d <= m, iota, n1), axis=1, keepdims=True)
        return m, im

    m1, i1 = extract(s)
    s = jnp.where(iota == i1, inf, s)
    m2, i2 = extract(s)
    s = jnp.where(iota == i2, inf, s)
    m3, i3 = extract(s)

    qn = jnp.sum(q * q, axis=1, keepdims=True)
    w1 = 1.0 / jnp.maximum(jnp.maximum(m1 + qn, 0.0), 1e-16)
    w2 = 1.0 / jnp.maximum(jnp.maximum(m2 + qn, 0.0), 1e-16)
    w3 = 1.0 / jnp.maximum(jnp.maximum(m3 + qn, 0.0), 1e-16)
    ws = w1 + w2 + w3
    idx_ref[:] = jnp.concatenate([i1, i2, i3], axis=1)
    wn_ref[:] = jnp.concatenate([w1 / ws, w2 / ws, w3 / ws], axis=1)


def _knn_call(ps_pad, posT):
    n2 = ps_pad.shape[0]
    n1 = posT.shape[1]
    r = 512
    return pl.pallas_call(
        functools.partial(_knn_body, n1=n1),
        grid=(n2 // r,),
        in_specs=[
            pl.BlockSpec((r, 8), lambda i: (i, 0)),
            pl.BlockSpec((8, n1), lambda i: (0, 0)),
        ],
        out_specs=[
            pl.BlockSpec((r, 3), lambda i: (i, 0)),
            pl.BlockSpec((r, 3), lambda i: (i, 0)),
        ],
        out_shape=[
            jax.ShapeDtypeStruct((n2, 3), jnp.int32),
            jax.ShapeDtypeStruct((n2, 3), jnp.float32),
        ],
    )(ps_pad, posT)


_QC = 128
_NBUF = 3


def _gather_call(x, idx2d, k_nn):
    n_chunk_rows, qc = idx2d.shape
    n_rows = n_chunk_rows * qc
    d = x.shape[1]
    info = plsc.get_sparse_core_info()
    nw = info.num_cores * info.num_subcores
    cpk = (n_chunk_rows // k_nn) // nw

    mesh = plsc.VectorSubcoreMesh(core_axis_name="c", subcore_axis_name="s")

    @functools.partial(
        pl.kernel,
        mesh=mesh,
        out_type=jax.ShapeDtypeStruct((n_rows, d), jnp.float32),
        scratch_types=(
            [pltpu.VMEM((k_nn * cpk, qc), jnp.int32)]
            + [pltpu.VMEM((qc, d), jnp.float32) for _ in range(_NBUF)]
            + [pltpu.SemaphoreType.DMA for _ in range(_NBUF)]
        ),
    )
    def gather_kernel(x_hbm, idx_hbm, out_hbm, idx_v, *bufs_and_sems):
        rows = bufs_and_sems[:_NBUF]
        sems = bufs_and_sems[_NBUF:]
        wid = lax.axis_index("s") * info.num_cores + lax.axis_index("c")
        for k in range(k_nn):
            pltpu.sync_copy(
                idx_hbm.at[pl.ds(k * (n_chunk_rows // k_nn) + cpk * wid, cpk)],
                idx_v.at[pl.ds(k * cpk, cpk)])
        n_ch = k_nn * cpk
        kpt = n_chunk_rows // k_nn

        def chunk_row(i):
            k, c = divmod(i, cpk)
            return k * kpt + cpk * wid + c

        for j in range(min(_NBUF, n_ch)):
            pltpu.async_copy(x_hbm.at[idx_v.at[j]], rows[j], sems[j])
        for i in range(n_ch):
            b = i % _NBUF
            pltpu.make_async_copy(x_hbm.at[idx_v.at[i]], rows[b],
                                  sems[b]).wait()
            pltpu.sync_copy(rows[b], out_hbm.at[pl.ds(chunk_row(i) * qc, qc)])
            nxt = i + _NBUF
            if nxt < n_ch:
                pltpu.async_copy(x_hbm.at[idx_v.at[nxt]], rows[b], sems[b])

    return gather_kernel(x, idx2d)


def _mlp_body(f_ref, wn_ref, xs_ref, w1_ref, b1_ref, w2_ref, b2_ref, out_ref):
    d_in = f_ref.shape[2]
    wn = wn_ref[:]
    y = (f_ref[0] * wn[:, 0:1]
         + f_ref[1] * wn[:, 1:2]
         + f_ref[2] * wn[:, 2:3])
    dn = (((1,), (0,)), ((), ()))
    h = jnp.maximum(
        lax.dot_general(y, w1_ref[0:d_in, :], dn,
                        preferred_element_type=jnp.float32)
        + lax.dot_general(xs_ref[:], w1_ref[d_in:, :], dn,
                          preferred_element_type=jnp.float32)
        + b1_ref[:], 0.0)
    out_ref[:] = jnp.maximum(
        lax.dot_general(h, w2_ref[:], dn, preferred_element_type=jnp.float32)
        + b2_ref[:], 0.0)


def _mlp_call(feats, wn, x_skip, w1, b1r, w2, b2r):
    n2 = feats.shape[1]
    d_in = feats.shape[2]
    d_skip = x_skip.shape[1]
    d_tot = w1.shape[0]
    d_hid = w1.shape[1]
    d_out = w2.shape[1]
    r = 512
    return pl.pallas_call(
        _mlp_body,
        grid=(n2 // r,),
        in_specs=[
            pl.BlockSpec((3, r, d_in), lambda i: (0, i, 0)),
            pl.BlockSpec((r, 3), lambda i: (i, 0)),
            pl.BlockSpec((r, d_skip), lambda i: (i, 0)),
            pl.BlockSpec((d_tot, d_hid), lambda i: (0, 0)),
            pl.BlockSpec((1, d_hid), lambda i: (0, 0)),
            pl.BlockSpec((d_hid, d_out), lambda i: (0, 0)),
            pl.BlockSpec((1, d_out), lambda i: (0, 0)),
        ],
        out_specs=pl.BlockSpec((r, d_out), lambda i: (i, 0)),
        out_shape=jax.ShapeDtypeStruct((n2, d_out), jnp.float32),
    )(feats, wn, x_skip, w1, b1r, w2, b2r)


def kernel(x, pos, batch, x_skip, pos_skip, batch_skip, W1, b1, W2, b2):
    n1, d_in = x.shape
    n2 = pos_skip.shape[0]
    ps_pad = jnp.pad(pos_skip, ((0, 0), (0, 8 - pos_skip.shape[1])))
    posT = jnp.pad(pos, ((0, 0), (0, 8 - pos.shape[1]))).T
    idx, wn = _knn_call(ps_pad, posT)
    k_nn = idx.shape[1]
    idx2d = idx.T.reshape(-1, _QC)
    feats = _gather_call(x, idx2d, k_nn)
    out = _mlp_call(feats.reshape(k_nn, n2, d_in), wn, x_skip,
                    W1, b1.reshape(1, -1), W2, b2.reshape(1, -1))
    return (out, pos_skip, batch_skip)


# speedup vs baseline: 14.3259x; 1.0150x over previous
import functools

import jax
import jax.numpy as jnp
from jax import lax
from jax.experimental import pallas as pl
from jax.experimental.pallas import tpu as pltpu
from jax.experimental.pallas import tpu_sc as plsc


def _knn_body(ps_ref, pos_ref, idx_ref, wn_ref, *, n1):
    q = ps_ref[:]
    kT = pos_ref[:]
    kn = jnp.sum(kT * kT, axis=0, keepdims=True)
    dt = lax.dot_general(q, kT, (((1,), (0,)), ((), ())),
                         preferred_element_type=jnp.float32)
    s = kn - 2.0 * dt
    iota = lax.broadcasted_iota(jnp.int32, s.shape, 1)
    inf = jnp.float32(jnp.inf)

    def extract(d):
        m = jnp.min(d, axis=1, keepdims=True)
        im = jnp.min(jnp.where(d <= m, iota, n1), axis=1, keepdims=True)
        return m, im

    m1, i1 = extract(s)
    s = jnp.where(iota == i1, inf, s)
    m2, i2 = extract(s)
    s = jnp.where(iota == i2, inf, s)
    m3, i3 = extract(s)

    qn = jnp.sum(q * q, axis=1, keepdims=True)
    w1 = 1.0 / jnp.maximum(jnp.maximum(m1 + qn, 0.0), 1e-16)
    w2 = 1.0 / jnp.maximum(jnp.maximum(m2 + qn, 0.0), 1e-16)
    w3 = 1.0 / jnp.maximum(jnp.maximum(m3 + qn, 0.0), 1e-16)
    ws = w1 + w2 + w3
    idx_ref[:] = jnp.concatenate([i1, i2, i3], axis=1)
    wn_ref[:] = jnp.concatenate([w1 / ws, w2 / ws, w3 / ws], axis=1)


def _knn_call(ps_pad, posT):
    n2 = ps_pad.shape[0]
    n1 = posT.shape[1]
    r = 512
    return pl.pallas_call(
        functools.partial(_knn_body, n1=n1),
        grid=(n2 // r,),
        in_specs=[
            pl.BlockSpec((r, 8), lambda i: (i, 0)),
            pl.BlockSpec((8, n1), lambda i: (0, 0)),
        ],
        out_specs=[
            pl.BlockSpec((r, 3), lambda i: (i, 0)),
            pl.BlockSpec((r, 3), lambda i: (i, 0)),
        ],
        out_shape=[
            jax.ShapeDtypeStruct((n2, 3), jnp.int32),
            jax.ShapeDtypeStruct((n2, 3), jnp.float32),
        ],
    )(ps_pad, posT)


_QC = 128
_NBUF = 3


def _gather_call(x, idx2d, k_nn):
    n_chunk_rows, qc = idx2d.shape
    n_rows = n_chunk_rows * qc
    d = x.shape[1]
    info = plsc.get_sparse_core_info()
    nw = info.num_cores * info.num_subcores
    cpk = (n_chunk_rows // k_nn) // nw

    mesh = plsc.VectorSubcoreMesh(core_axis_name="c", subcore_axis_name="s")

    @functools.partial(
        pl.kernel,
        mesh=mesh,
        out_type=jax.ShapeDtypeStruct((n_rows, d), jnp.float32),
        scratch_types=(
            [pltpu.VMEM((k_nn * cpk, qc), jnp.int32)]
            + [pltpu.VMEM((qc, d), jnp.float32) for _ in range(_NBUF)]
            + [pltpu.SemaphoreType.DMA for _ in range(_NBUF)]
        ),
    )
    def gather_kernel(x_hbm, idx_hbm, out_hbm, idx_v, *bufs_and_sems):
        rows = bufs_and_sems[:_NBUF]
        sems = bufs_and_sems[_NBUF:]
        wid = lax.axis_index("s") * info.num_cores + lax.axis_index("c")
        for k in range(k_nn):
            pltpu.sync_copy(
                idx_hbm.at[pl.ds(k * (n_chunk_rows // k_nn) + cpk * wid, cpk)],
                idx_v.at[pl.ds(k * cpk, cpk)])
        n_ch = k_nn * cpk
        kpt = n_chunk_rows // k_nn

        def chunk_row(i):
            k, c = divmod(i, cpk)
            return k * kpt + cpk * wid + c

        for j in range(min(_NBUF, n_ch)):
            pltpu.async_copy(x_hbm.at[idx_v.at[j]], rows[j], sems[j])
        for i in range(n_ch):
            b = i % _NBUF
            pltpu.make_async_copy(x_hbm.at[idx_v.at[i]], rows[b],
                                  sems[b]).wait()
            pltpu.sync_copy(rows[b], out_hbm.at[pl.ds(chunk_row(i) * qc, qc)])
            nxt = i + _NBUF
            if nxt < n_ch:
                pltpu.async_copy(x_hbm.at[idx_v.at[nxt]], rows[b], sems[b])

    return gather_kernel(x, idx2d)


def _mlp_body(f_ref, wn_ref, xs_ref, w1_ref, b1_ref, w2_ref, b2_ref, out_ref):
    d_in = f_ref.shape[2]
    wn = wn_ref[:]
    y = (f_ref[0] * wn[:, 0:1]
         + f_ref[1] * wn[:, 1:2]
         + f_ref[2] * wn[:, 2:3])
    dn = (((1,), (0,)), ((), ()))
    h = jnp.maximum(
        lax.dot_general(y, w1_ref[0:d_in, :], dn,
                        preferred_element_type=jnp.float32)
        + lax.dot_general(xs_ref[:], w1_ref[d_in:, :], dn,
                          preferred_element_type=jnp.float32)
        + b1_ref[:], 0.0)
    out_ref[:] = jnp.maximum(
        lax.dot_general(h, w2_ref[:], dn, preferred_element_type=jnp.float32)
        + b2_ref[:], 0.0)


def _mlp_call(feats, wn, x_skip, w1, b1r, w2, b2r):
    n2 = feats.shape[1]
    d_in = feats.shape[2]
    d_skip = x_skip.shape[1]
    d_tot = w1.shape[0]
    d_hid = w1.shape[1]
    d_out = w2.shape[1]
    r = 512
    return pl.pallas_call(
        _mlp_body,
        grid=(n2 // r,),
        in_specs=[
            pl.BlockSpec((3, r, d_in), lambda i: (0, i, 0)),
            pl.BlockSpec((r, 3), lambda i: (i, 0)),
            pl.BlockSpec((r, d_skip), lambda i: (i, 0)),
            pl.BlockSpec((d_tot, d_hid), lambda i: (0, 0)),
            pl.BlockSpec((1, d_hid), lambda i: (0, 0)),
            pl.BlockSpec((d_hid, d_out), lambda i: (0, 0)),
            pl.BlockSpec((1, d_out), lambda i: (0, 0)),
        ],
        out_specs=pl.BlockSpec((r, d_out), lambda i: (i, 0)),
        out_shape=jax.ShapeDtypeStruct((n2, d_out), jnp.float32),
    )(feats, wn, x_skip, w1, b1r, w2, b2r)


def kernel(x, pos, batch, x_skip, pos_skip, batch_skip, W1, b1, W2, b2):
    n1, d_in = x.shape
    n2 = pos_skip.shape[0]
    posT = jnp.pad(pos, ((0, 0), (0, 8 - pos.shape[1]))).T
    b1r = b1.reshape(1, -1)
    b2r = b2.reshape(1, -1)
    nh = n2 // 2
    outs = []
    for h in range(2):
        ps = pos_skip[h * nh:(h + 1) * nh]
        xs = x_skip[h * nh:(h + 1) * nh]
        ps_pad = jnp.pad(ps, ((0, 0), (0, 8 - ps.shape[1])))
        idx, wn = _knn_call(ps_pad, posT)
        k_nn = idx.shape[1]
        idx2d = idx.T.reshape(-1, _QC)
        feats = _gather_call(x, idx2d, k_nn)
        outs.append(_mlp_call(feats.reshape(k_nn, nh, d_in), wn, xs,
                              W1, b1r, W2, b2r))
    return (jnp.concatenate(outs, axis=0), pos_skip, batch_skip)


# speedup vs baseline: 16.2654x; 1.1354x over previous
import functools

import jax
import jax.numpy as jnp
from jax import lax
from jax.experimental import pallas as pl
from jax.experimental.pallas import tpu as pltpu
from jax.experimental.pallas import tpu_sc as plsc


def _knn_body(ps_ref, pos_ref, idx_ref, wn_ref, *, n1):
    q = ps_ref[:]
    kT = pos_ref[:]
    kn = jnp.sum(kT * kT, axis=0, keepdims=True)
    dt = lax.dot_general(q, kT, (((1,), (0,)), ((), ())),
                         preferred_element_type=jnp.float32)
    r = q.shape[0]
    inf = jnp.float32(jnp.inf)
    rb, cb = 64, 128
    lane = lax.broadcasted_iota(jnp.int32, (rb, cb), 1)
    s_parts, j_parts = [], []
    for r0 in range(0, r, rb):
        s1 = s2 = s3 = jnp.full((rb, cb), inf, jnp.float32)
        i1 = i2 = i3 = jnp.zeros((rb, cb), jnp.int32)
        for c0 in range(0, n1, cb):
            v = kn[:, c0:c0 + cb] - 2.0 * dt[r0:r0 + rb, c0:c0 + cb]
            jv = lane + c0
            c1 = v < s1
            c2 = v < s2
            c3 = v < s3
            s3 = jnp.where(c2, s2, jnp.where(c3, v, s3))
            i3 = jnp.where(c2, i2, jnp.where(c3, jv, i3))
            s2 = jnp.where(c1, s1, jnp.where(c2, v, s2))
            i2 = jnp.where(c1, i1, jnp.where(c2, jv, i2))
            s1 = jnp.where(c1, v, s1)
            i1 = jnp.where(c1, jv, i1)
        s_parts.append(jnp.concatenate([s1, s2, s3], axis=1))
        j_parts.append(jnp.concatenate([i1, i2, i3], axis=1))
    S = jnp.concatenate(s_parts, axis=0)
    J = jnp.concatenate(j_parts, axis=0)

    def extract(S, J):
        m = jnp.min(S, axis=1, keepdims=True)
        im = jnp.min(jnp.where(S <= m, J, n1), axis=1, keepdims=True)
        return m, im

    m1, i1 = extract(S, J)
    S = jnp.where(J == i1, inf, S)
    m2, i2 = extract(S, J)
    S = jnp.where(J == i2, inf, S)
    m3, i3 = extract(S, J)

    qn = jnp.sum(q * q, axis=1, keepdims=True)
    w1 = 1.0 / jnp.maximum(jnp.maximum(m1 + qn, 0.0), 1e-16)
    w2 = 1.0 / jnp.maximum(jnp.maximum(m2 + qn, 0.0), 1e-16)
    w3 = 1.0 / jnp.maximum(jnp.maximum(m3 + qn, 0.0), 1e-16)
    ws = w1 + w2 + w3
    idx_ref[:] = jnp.concatenate([i1, i2, i3], axis=1)
    wn_ref[:] = jnp.concatenate([w1 / ws, w2 / ws, w3 / ws], axis=1)


def _knn_call(ps_pad, posT):
    n2 = ps_pad.shape[0]
    n1 = posT.shape[1]
    r = 512
    return pl.pallas_call(
        functools.partial(_knn_body, n1=n1),
        grid=(n2 // r,),
        in_specs=[
            pl.BlockSpec((r, 8), lambda i: (i, 0)),
            pl.BlockSpec((8, n1), lambda i: (0, 0)),
        ],
        out_specs=[
            pl.BlockSpec((r, 3), lambda i: (i, 0)),
            pl.BlockSpec((r, 3), lambda i: (i, 0)),
        ],
        out_shape=[
            jax.ShapeDtypeStruct((n2, 3), jnp.int32),
            jax.ShapeDtypeStruct((n2, 3), jnp.float32),
        ],
    )(ps_pad, posT)


_QC = 128
_NBUF = 3


def _gather_call(x, idx2d, k_nn):
    n_chunk_rows, qc = idx2d.shape
    n_rows = n_chunk_rows * qc
    d = x.shape[1]
    info = plsc.get_sparse_core_info()
    nw = info.num_cores * info.num_subcores
    cpk = (n_chunk_rows // k_nn) // nw

    mesh = plsc.VectorSubcoreMesh(core_axis_name="c", subcore_axis_name="s")

    @functools.partial(
        pl.kernel,
        mesh=mesh,
        out_type=jax.ShapeDtypeStruct((n_rows, d), jnp.float32),
        scratch_types=(
            [pltpu.VMEM((k_nn * cpk, qc), jnp.int32)]
            + [pltpu.VMEM((qc, d), jnp.float32) for _ in range(_NBUF)]
            + [pltpu.SemaphoreType.DMA for _ in range(_NBUF)]
        ),
    )
    def gather_kernel(x_hbm, idx_hbm, out_hbm, idx_v, *bufs_and_sems):
        rows = bufs_and_sems[:_NBUF]
        sems = bufs_and_sems[_NBUF:]
        wid = lax.axis_index("s") * info.num_cores + lax.axis_index("c")
        for k in range(k_nn):
            pltpu.sync_copy(
                idx_hbm.at[pl.ds(k * (n_chunk_rows // k_nn) + cpk * wid, cpk)],
                idx_v.at[pl.ds(k * cpk, cpk)])
        n_ch = k_nn * cpk
        kpt = n_chunk_rows // k_nn

        def chunk_row(i):
            k, c = divmod(i, cpk)
            return k * kpt + cpk * wid + c

        for j in range(min(_NBUF, n_ch)):
            pltpu.async_copy(x_hbm.at[idx_v.at[j]], rows[j], sems[j])
        for i in range(n_ch):
            b = i % _NBUF
            pltpu.make_async_copy(x_hbm.at[idx_v.at[i]], rows[b],
                                  sems[b]).wait()
            pltpu.sync_copy(rows[b], out_hbm.at[pl.ds(chunk_row(i) * qc, qc)])
            nxt = i + _NBUF
            if nxt < n_ch:
                pltpu.async_copy(x_hbm.at[idx_v.at[nxt]], rows[b], sems[b])

    return gather_kernel(x, idx2d)


def _mlp_body(f_ref, wn_ref, xs_ref, w1_ref, b1_ref, w2_ref, b2_ref, out_ref):
    d_in = f_ref.shape[2]
    wn = wn_ref[:]
    y = (f_ref[0] * wn[:, 0:1]
         + f_ref[1] * wn[:, 1:2]
         + f_ref[2] * wn[:, 2:3])
    dn = (((1,), (0,)), ((), ()))
    h = jnp.maximum(
        lax.dot_general(y, w1_ref[0:d_in, :], dn,
                        preferred_element_type=jnp.float32)
        + lax.dot_general(xs_ref[:], w1_ref[d_in:, :], dn,
                          preferred_element_type=jnp.float32)
        + b1_ref[:], 0.0)
    out_ref[:] = jnp.maximum(
        lax.dot_general(h, w2_ref[:], dn, preferred_element_type=jnp.float32)
        + b2_ref[:], 0.0)


def _mlp_call(feats, wn, x_skip, w1, b1r, w2, b2r):
    n2 = feats.shape[1]
    d_in = feats.shape[2]
    d_skip = x_skip.shape[1]
    d_tot = w1.shape[0]
    d_hid = w1.shape[1]
    d_out = w2.shape[1]
    r = 512
    return pl.pallas_call(
        _mlp_body,
        grid=(n2 // r,),
        in_specs=[
            pl.BlockSpec((3, r, d_in), lambda i: (0, i, 0)),
            pl.BlockSpec((r, 3), lambda i: (i, 0)),
            pl.BlockSpec((r, d_skip), lambda i: (i, 0)),
            pl.BlockSpec((d_tot, d_hid), lambda i: (0, 0)),
            pl.BlockSpec((1, d_hid), lambda i: (0, 0)),
            pl.BlockSpec((d_hid, d_out), lambda i: (0, 0)),
            pl.BlockSpec((1, d_out), lambda i: (0, 0)),
        ],
        out_specs=pl.BlockSpec((r, d_out), lambda i: (i, 0)),
        out_shape=jax.ShapeDtypeStruct((n2, d_out), jnp.float32),
    )(feats, wn, x_skip, w1, b1r, w2, b2r)


def kernel(x, pos, batch, x_skip, pos_skip, batch_skip, W1, b1, W2, b2):
    n1, d_in = x.shape
    n2 = pos_skip.shape[0]
    posT = jnp.pad(pos, ((0, 0), (0, 8 - pos.shape[1]))).T
    b1r = b1.reshape(1, -1)
    b2r = b2.reshape(1, -1)
    nh = n2 // 2
    outs = []
    for h in range(2):
        ps = pos_skip[h * nh:(h + 1) * nh]
        xs = x_skip[h * nh:(h + 1) * nh]
        ps_pad = jnp.pad(ps, ((0, 0), (0, 8 - ps.shape[1])))
        idx, wn = _knn_call(ps_pad, posT)
        k_nn = idx.shape[1]
        idx2d = idx.T.reshape(-1, _QC)
        feats = _gather_call(x, idx2d, k_nn)
        outs.append(_mlp_call(feats.reshape(k_nn, nh, d_in), wn, xs,
                              W1, b1r, W2, b2r))
    return (jnp.concatenate(outs, axis=0), pos_skip, batch_skip)


# speedup vs baseline: 16.7320x; 1.0287x over previous
import functools

import jax
import jax.numpy as jnp
from jax import lax
from jax.experimental import pallas as pl
from jax.experimental.pallas import tpu as pltpu
from jax.experimental.pallas import tpu_sc as plsc


def _knn_body(ps_ref, pos_ref, idx_ref, wn_ref, *, n1):
    q = ps_ref[:]
    kT = pos_ref[:]
    kn = jnp.sum(kT * kT, axis=0, keepdims=True)
    dt2 = lax.dot_general(q * -2.0, kT, (((1,), (0,)), ((), ())),
                          preferred_element_type=jnp.float32)
    r = q.shape[0]
    inf = jnp.float32(jnp.inf)
    rb, cb = 64, 128
    s_parts, j_parts = [], []
    for r0 in range(0, r, rb):
        s1 = s2 = s3 = jnp.full((rb, cb), inf, jnp.float32)
        i1 = i2 = i3 = jnp.zeros((rb, cb), jnp.int32)
        for c0 in range(0, n1, cb):
            v = kn[:, c0:c0 + cb] + dt2[r0:r0 + rb, c0:c0 + cb]
            jv = jnp.full((rb, cb), c0, jnp.int32)
            c1 = v < s1
            c2 = v < s2
            c3 = v < s3
            s3 = jnp.where(c2, s2, jnp.where(c3, v, s3))
            i3 = jnp.where(c2, i2, jnp.where(c3, jv, i3))
            s2 = jnp.where(c1, s1, jnp.where(c2, v, s2))
            i2 = jnp.where(c1, i1, jnp.where(c2, jv, i2))
            s1 = jnp.where(c1, v, s1)
            i1 = jnp.where(c1, jv, i1)
        s_parts.append(jnp.concatenate([s1, s2, s3], axis=1))
        j_parts.append(jnp.concatenate([i1, i2, i3], axis=1))
    S = jnp.concatenate(s_parts, axis=0)
    lane3 = jnp.remainder(
        lax.broadcasted_iota(jnp.int32, S.shape, 1), cb)
    J = jnp.concatenate(j_parts, axis=0) + lane3

    def extract(S, J):
        m = jnp.min(S, axis=1, keepdims=True)
        im = jnp.min(jnp.where(S <= m, J, n1), axis=1, keepdims=True)
        return m, im

    m1, i1 = extract(S, J)
    S = jnp.where(J == i1, inf, S)
    m2, i2 = extract(S, J)
    S = jnp.where(J == i2, inf, S)
    m3, i3 = extract(S, J)

    qn = jnp.sum(q * q, axis=1, keepdims=True)
    w1 = 1.0 / jnp.maximum(jnp.maximum(m1 + qn, 0.0), 1e-16)
    w2 = 1.0 / jnp.maximum(jnp.maximum(m2 + qn, 0.0), 1e-16)
    w3 = 1.0 / jnp.maximum(jnp.maximum(m3 + qn, 0.0), 1e-16)
    ws = w1 + w2 + w3
    idx_ref[:] = jnp.concatenate([i1, i2, i3], axis=1)
    wn_ref[:] = jnp.concatenate([w1 / ws, w2 / ws, w3 / ws], axis=1)


def _knn_call(ps_pad, posT):
    n2 = ps_pad.shape[0]
    n1 = posT.shape[1]
    r = 512
    return pl.pallas_call(
        functools.partial(_knn_body, n1=n1),
        grid=(n2 // r,),
        in_specs=[
            pl.BlockSpec((r, 8), lambda i: (i, 0)),
            pl.BlockSpec((8, n1), lambda i: (0, 0)),
        ],
        out_specs=[
            pl.BlockSpec((r, 3), lambda i: (i, 0)),
            pl.BlockSpec((r, 3), lambda i: (i, 0)),
        ],
        out_shape=[
            jax.ShapeDtypeStruct((n2, 3), jnp.int32),
            jax.ShapeDtypeStruct((n2, 3), jnp.float32),
        ],
    )(ps_pad, posT)


_QC = 128
_NBUF = 3


def _gather_call(x, idx2d, k_nn):
    n_chunk_rows, qc = idx2d.shape
    n_rows = n_chunk_rows * qc
    d = x.shape[1]
    info = plsc.get_sparse_core_info()
    nw = info.num_cores * info.num_subcores
    cpk = (n_chunk_rows // k_nn) // nw

    mesh = plsc.VectorSubcoreMesh(core_axis_name="c", subcore_axis_name="s")

    @functools.partial(
        pl.kernel,
        mesh=mesh,
        out_type=jax.ShapeDtypeStruct((n_rows, d), jnp.float32),
        scratch_types=(
            [pltpu.VMEM((k_nn * cpk, qc), jnp.int32)]
            + [pltpu.VMEM((qc, d), jnp.float32) for _ in range(_NBUF)]
            + [pltpu.SemaphoreType.DMA for _ in range(_NBUF)]
        ),
    )
    def gather_kernel(x_hbm, idx_hbm, out_hbm, idx_v, *bufs_and_sems):
        rows = bufs_and_sems[:_NBUF]
        sems = bufs_and_sems[_NBUF:]
        wid = lax.axis_index("s") * info.num_cores + lax.axis_index("c")
        for k in range(k_nn):
            pltpu.sync_copy(
                idx_hbm.at[pl.ds(k * (n_chunk_rows // k_nn) + cpk * wid, cpk)],
                idx_v.at[pl.ds(k * cpk, cpk)])
        n_ch = k_nn * cpk
        kpt = n_chunk_rows // k_nn

        def chunk_row(i):
            k, c = divmod(i, cpk)
            return k * kpt + cpk * wid + c

        for j in range(min(_NBUF, n_ch)):
            pltpu.async_copy(x_hbm.at[idx_v.at[j]], rows[j], sems[j])
        for i in range(n_ch):
            b = i % _NBUF
            pltpu.make_async_copy(x_hbm.at[idx_v.at[i]], rows[b],
                                  sems[b]).wait()
            pltpu.sync_copy(rows[b], out_hbm.at[pl.ds(chunk_row(i) * qc, qc)])
            nxt = i + _NBUF
            if nxt < n_ch:
                pltpu.async_copy(x_hbm.at[idx_v.at[nxt]], rows[b], sems[b])

    return gather_kernel(x, idx2d)


def _mlp_body(f_ref, wn_ref, xs_ref, w1_ref, b1_ref, w2_ref, b2_ref, out_ref):
    d_in = f_ref.shape[2]
    wn = wn_ref[:]
    y = (f_ref[0] * wn[:, 0:1]
         + f_ref[1] * wn[:, 1:2]
         + f_ref[2] * wn[:, 2:3])
    dn = (((1,), (0,)), ((), ()))
    h = jnp.maximum(
        lax.dot_general(y, w1_ref[0:d_in, :], dn,
                        preferred_element_type=jnp.float32)
        + lax.dot_general(xs_ref[:], w1_ref[d_in:, :], dn,
                          preferred_element_type=jnp.float32)
        + b1_ref[:], 0.0)
    out_ref[:] = jnp.maximum(
        lax.dot_general(h, w2_ref[:], dn, preferred_element_type=jnp.float32)
        + b2_ref[:], 0.0)


def _mlp_call(feats, wn, x_skip, w1, b1r, w2, b2r):
    n2 = feats.shape[1]
    d_in = feats.shape[2]
    d_skip = x_skip.shape[1]
    d_tot = w1.shape[0]
    d_hid = w1.shape[1]
    d_out = w2.shape[1]
    r = 512
    return pl.pallas_call(
        _mlp_body,
        grid=(n2 // r,),
        in_specs=[
            pl.BlockSpec((3, r, d_in), lambda i: (0, i, 0)),
            pl.BlockSpec((r, 3), lambda i: (i, 0)),
            pl.BlockSpec((r, d_skip), lambda i: (i, 0)),
            pl.BlockSpec((d_tot, d_hid), lambda i: (0, 0)),
            pl.BlockSpec((1, d_hid), lambda i: (0, 0)),
            pl.BlockSpec((d_hid, d_out), lambda i: (0, 0)),
            pl.BlockSpec((1, d_out), lambda i: (0, 0)),
        ],
        out_specs=pl.BlockSpec((r, d_out), lambda i: (i, 0)),
        out_shape=jax.ShapeDtypeStruct((n2, d_out), jnp.float32),
    )(feats, wn, x_skip, w1, b1r, w2, b2r)


def kernel(x, pos, batch, x_skip, pos_skip, batch_skip, W1, b1, W2, b2):
    n1, d_in = x.shape
    n2 = pos_skip.shape[0]
    posT = jnp.pad(pos, ((0, 0), (0, 8 - pos.shape[1]))).T
    b1r = b1.reshape(1, -1)
    b2r = b2.reshape(1, -1)
    nh = n2 // 2
    outs = []
    for h in range(2):
        ps = pos_skip[h * nh:(h + 1) * nh]
        xs = x_skip[h * nh:(h + 1) * nh]
        ps_pad = jnp.pad(ps, ((0, 0), (0, 8 - ps.shape[1])))
        idx, wn = _knn_call(ps_pad, posT)
        k_nn = idx.shape[1]
        idx2d = idx.T.reshape(-1, _QC)
        feats = _gather_call(x, idx2d, k_nn)
        outs.append(_mlp_call(feats.reshape(k_nn, nh, d_in), wn, xs,
                              W1, b1r, W2, b2r))
    return (jnp.concatenate(outs, axis=0), pos_skip, batch_skip)
